# baseline (speedup 1.0000x reference)
import functools

import jax
import jax.numpy as jnp
from jax import lax
from jax.experimental import pallas as pl
from jax.experimental.pallas import tpu as pltpu
from jax.experimental.pallas import tpu_sc as plsc

BATCH = 16384
FIELDS = 26
EMBEDDING_SIZE = 32
CARD = 1000000

NC = 2
NS = 16
NW = NC * NS

D = EMBEDDING_SIZE
W4 = CARD // 4
BPW = BATCH // NW
CB = 128
NCB = BPW // CB
NCHUNK = FIELDS * NCB
NBUF = 4
L = 16
assert BPW * NW == BATCH and (NCHUNK - 4) % 2 == 0


def _gather_kernel(idxt_hbm, w4_hbm, out_hbm, idxt_v, g_v, rows4_v, fbuf_v,
                   gsem0, gsem1, osem0, osem1):
    wid = lax.axis_index("s") * NC + lax.axis_index("c")
    b0 = wid * BPW
    pltpu.sync_copy(idxt_hbm.at[:, pl.ds(b0, BPW)], idxt_v)

    gsems = (gsem0, gsem1)
    osems = (osem0, osem1)

    def prep_gather(j, slot):
        f = j // NCB
        cb = lax.rem(j, NCB)
        for t in range(CB // L):
            g_v[slot, pl.ds(t * L, L)] = lax.shift_right_logical(
                idxt_v[f, pl.ds(cb * CB + t * L, L)], 2)

    def gather_chunk(slot, par):
        return pltpu.make_async_copy(
            w4_hbm.at[g_v.at[slot]], rows4_v.at[slot], gsems[par])

    def out_chunk(j, eslot, par):
        f = j // NCB
        cb = lax.rem(j, NCB)
        return pltpu.make_async_copy(
            fbuf_v.at[eslot],
            out_hbm.at[f, :, pl.ds(b0 + cb * CB, CB)], osems[par])

    lanes = lax.iota(jnp.int32, L)

    def extract(j, slot, eslot):
        f = j // NCB
        cb = lax.rem(j, NCB)
        rslot = rows4_v.at[slot]

        def grp(t, _):
            o = t * L
            r = idxt_v[f, pl.ds(cb * CB + o, L)]
            col0 = lax.bitwise_and(r, 3) * D
            row = lanes + o
            for e in range(D):
                vals = plsc.load_gather(rslot, [row, col0 + e])
                fbuf_v[eslot, e, pl.ds(o, L)] = vals
            return 0

        lax.fori_loop(0, CB // L, grp, 0)

    for j in (0, 1):
        prep_gather(j, j)
        gather_chunk(j, j).start()

    for j in (0, 1):
        gather_chunk(j, j % 2).wait()
        prep_gather(j + 2, j + 2)
        gather_chunk(j + 2, j % 2).start()
        extract(j, j, j % 2)
        out_chunk(j, j % 2, j % 2).start()

    def step(j, par):
        slot = lax.rem(j, NBUF)
        gather_chunk(slot, par).wait()
        out_chunk(j - 2, par, par).wait()
        nslot = lax.rem(j + 2, NBUF)
        prep_gather(j + 2, nslot)
        gather_chunk(nslot, par).start()
        extract(j, slot, par)
        out_chunk(j, par, par).start()

    def body(i, _):
        j = 2 + 2 * i
        step(j, 0)
        step(j + 1, 1)
        return 0

    lax.fori_loop(0, (NCHUNK - 4) // 2, body, 0)

    for j in (NCHUNK - 2, NCHUNK - 1):
        gather_chunk(j % NBUF, j % 2).wait()
        out_chunk(j - 2, j % 2, j % 2).wait()
        extract(j, j % NBUF, j % 2)
        out_chunk(j, j % 2, j % 2).start()
    for j in (NCHUNK - 2, NCHUNK - 1):
        out_chunk(j, j % 2, j % 2).wait()


@jax.jit
def kernel(inputs, weight):
    idxt = inputs.astype(jnp.int32).T
    w4 = weight.reshape(W4, 4 * D)
    mesh = plsc.VectorSubcoreMesh(core_axis_name="c", subcore_axis_name="s")
    out = pl.kernel(
        _gather_kernel,
        out_type=jax.ShapeDtypeStruct((FIELDS, D, BATCH), jnp.float32),
        mesh=mesh,
        scratch_types=[
            pltpu.VMEM((FIELDS, BPW), jnp.int32),
            pltpu.VMEM((NBUF, CB), jnp.int32),
            pltpu.VMEM((NBUF, CB, 4 * D), jnp.float32),
            pltpu.VMEM((2, D, CB), jnp.float32),
            pltpu.SemaphoreType.DMA,
            pltpu.SemaphoreType.DMA,
            pltpu.SemaphoreType.DMA,
            pltpu.SemaphoreType.DMA,
        ],
        compiler_params=pltpu.CompilerParams(needs_layout_passes=False),
    )(idxt, w4)
    return out.transpose(2, 0, 1)
